# Initial kernel scaffold; baseline (speedup 1.0000x reference)
import jax
import jax.numpy as jnp
from jax.experimental import pallas as pl


def kernel(query, value, seed, rand_matrix):
    raise NotImplementedError("write your pallas kernel here")


# speedup vs baseline: 28.6847x; 28.6847x over previous
import math

import jax
import jax.numpy as jnp
from jax.experimental import pallas as pl
from jax.experimental.pallas import tpu as pltpu

_ROUNDS = 4
_BL = 64
_STR = 256


def _attn_kernel(qv_ref, hi_ref, oi_ref, qib_ref, sbb_ref, cqb_ref,
                 out_ref, g_ref, y_ref):
    L, dk2 = qv_ref.shape[1], qv_ref.shape[2]
    dk = dk2 // 2
    r = _ROUNDS
    bl = _BL
    nb = L // bl
    nstr = L // _STR
    rr = pl.program_id(1)
    iota_s = jax.lax.broadcasted_iota(jnp.int32, (_STR, L), 1)

    def look_back2(x):
        prev = jnp.concatenate([x[nb - 1:], x[:nb - 1]], axis=0)
        return jnp.concatenate([prev, x], axis=1)

    def look_back3(x):
        prev = jnp.concatenate([x[nb - 1:], x[:nb - 1]], axis=0)
        return jnp.concatenate([prev, x], axis=1)

    def gath(s, c):
        hi_s = hi_ref[0, 0, pl.ds(s * _STR, _STR), :]
        P = (hi_s == iota_s).astype(jnp.float32)
        g_ref[pl.ds(s * _STR, _STR), :] = jnp.dot(
            P, qv_ref[0], preferred_element_type=jnp.float32)
        return c

    jax.lax.fori_loop(0, nstr, gath, 0)

    G = g_ref[...]
    rq = G[:, :dk]
    rv = G[:, dk:]
    kn = rq / jnp.maximum(
        jnp.sqrt(jnp.sum(rq * rq, axis=1, keepdims=True)), 1e-12)
    rq3 = rq.reshape(nb, bl, dk)
    rv3 = rv.reshape(nb, bl, dk)
    kn3 = kn.reshape(nb, bl, dk)
    lbk = look_back3(kn3)
    lbv = look_back3(rv3)
    qk = jax.lax.dot_general(
        rq3, lbk, (((2,), (2,)), ((0,), (0,))),
        preferred_element_type=jnp.float32) * (1.0 / math.sqrt(dk))
    sb3 = sbb_ref[0, 0]
    qi3 = qib_ref[0, 0]
    lbsb = look_back2(sb3)
    lbqi = look_back2(qi3)
    qk = jnp.where(sb3[:, :, None] != lbsb[:, None, :], -1e9, qk)
    qk = jnp.where(qi3[:, :, None] < lbqi[:, None, :], -1e9, qk)
    qk = jnp.where(qi3[:, :, None] == lbqi[:, None, :], -1e5, qk)
    ck = jnp.zeros((nb, bl, 2 * bl), jnp.float32)
    for r2 in range(r):
        cq = cqb_ref[0, 0, r2 * nb:(r2 + 1) * nb, :]
        ckey = look_back2(cq)
        cqm1 = jnp.where(cq == 0, nb - 1, cq - 1)
        hit = (ckey[:, None, :] == cq[:, :, None]) | (
            ckey[:, None, :] == cqm1[:, :, None])
        ck += hit.astype(jnp.float32)
    m = jnp.max(qk, axis=2)
    e = jnp.exp(qk - m[:, :, None])
    s = jnp.sum(e, axis=2)
    lse = m + jnp.log(s)
    sm = e / (s[:, :, None] * ck)
    att = jax.lax.dot_general(
        sm, lbv, (((2,), (1,)), ((0,), (0,))),
        preferred_element_type=jnp.float32)
    lm = jnp.max(lse, keepdims=True)
    w = jnp.exp(lse - lm)
    w = w / jnp.sum(w, keepdims=True)
    y_ref[...] = (att * w[:, :, None]).reshape(L, dk)

    def unsort(s, c):
        oi_s = oi_ref[0, 0, pl.ds(s * _STR, _STR), :]
        Q = (oi_s == iota_s).astype(jnp.float32)
        contrib = jnp.dot(
            Q, y_ref[...], preferred_element_type=jnp.float32)
        prev = out_ref[0, pl.ds(s * _STR, _STR), :]
        out_ref[0, pl.ds(s * _STR, _STR), :] = jnp.where(
            rr == 0, contrib, prev + contrib)
        return c

    jax.lax.fori_loop(0, nstr, unsort, 0)


def kernel(query, value, seed, rand_matrix):
    B, L, dk = query.shape
    r = rand_matrix.shape[2]
    bl = _BL
    nb = L // bl
    qn = query / jnp.maximum(
        jnp.sqrt(jnp.sum(query * query, axis=-1, keepdims=True)), 1e-12)
    rm = rand_matrix / jnp.sqrt(
        jnp.sum(rand_matrix * rand_matrix, axis=1, keepdims=True))
    mm = jnp.einsum('bij,bjkl->bikl', qn, rm)
    h = jnp.argmax(jnp.concatenate([mm, -mm], -1), -1).astype(jnp.int32)
    h = h * L + jnp.arange(L, dtype=jnp.int32)[None, :, None]
    sorted_h = jnp.sort(h, axis=1)
    hi = (sorted_h % L).astype(jnp.int32)
    sb = (sorted_h // L).astype(jnp.int32)
    oi = jnp.argsort(hi, axis=1).astype(jnp.int32)
    ch = oi // bl
    chq = jnp.take_along_axis(
        jnp.broadcast_to(ch[:, :, None, :], (B, L, r, r)),
        jnp.broadcast_to(hi[:, :, :, None], (B, L, r, r)), axis=1)
    hi_t = jnp.transpose(hi, (0, 2, 1))[:, :, :, None]
    oi_t = jnp.transpose(oi, (0, 2, 1))[:, :, :, None]
    qib = jnp.transpose(hi, (0, 2, 1)).reshape(B, r, nb, bl)
    sbb = jnp.transpose(sb, (0, 2, 1)).reshape(B, r, nb, bl)
    cqb = jnp.transpose(chq, (0, 2, 3, 1)).reshape(B, r, r * nb, bl)
    qv = jnp.concatenate([query, value], axis=-1)

    grid = (B, r)
    out = pl.pallas_call(
        _attn_kernel,
        grid=grid,
        in_specs=[
            pl.BlockSpec((1, L, 2 * dk), lambda b, rr: (b, 0, 0)),
            pl.BlockSpec((1, 1, L, 1), lambda b, rr: (b, rr, 0, 0)),
            pl.BlockSpec((1, 1, L, 1), lambda b, rr: (b, rr, 0, 0)),
            pl.BlockSpec((1, 1, nb, bl), lambda b, rr: (b, rr, 0, 0)),
            pl.BlockSpec((1, 1, nb, bl), lambda b, rr: (b, rr, 0, 0)),
            pl.BlockSpec((1, 1, r * nb, bl), lambda b, rr: (b, rr, 0, 0)),
        ],
        out_specs=pl.BlockSpec((1, L, dk), lambda b, rr: (b, 0, 0)),
        out_shape=jax.ShapeDtypeStruct((B, L, dk), jnp.float32),
        scratch_shapes=[
            pltpu.VMEM((L, 2 * dk), jnp.float32),
            pltpu.VMEM((L, dk), jnp.float32),
        ],
    )(qv, hi_t, oi_t, qib, sbb, cqb)
    return out


# speedup vs baseline: 41.3619x; 1.4419x over previous
import functools
import math

import jax
import jax.numpy as jnp
from jax.experimental import pallas as pl
from jax.experimental.pallas import tpu as pltpu
from jax.experimental.pallas import tpu_sc as plsc

_ROUNDS = 4
_BL = 64


def _sc_gather(table, idx, out_dim):
    info = plsc.get_sparse_core_info()
    nw = info.num_cores * info.num_subcores
    M = idx.shape[0]
    m_per_w = M // nw
    chunk = 65536 // out_dim
    n_chunks = m_per_w // chunk
    mesh = plsc.VectorSubcoreMesh(core_axis_name="c", subcore_axis_name="s")

    @functools.partial(
        pl.kernel, mesh=mesh,
        out_type=jax.ShapeDtypeStruct((M, out_dim), jnp.float32),
        scratch_types=[
            pltpu.VMEM((chunk,), jnp.int32),
            pltpu.VMEM((chunk, out_dim), jnp.float32),
            pltpu.SemaphoreType.DMA,
        ],
    )
    def k(table_hbm, idx_hbm, out_hbm, idx_v, rows_v, sem):
        wid = jax.lax.axis_index("s") * info.num_cores + jax.lax.axis_index("c")
        base = wid * m_per_w
        for j in range(n_chunks):
            off = base + j * chunk
            pltpu.sync_copy(idx_hbm.at[pl.ds(off, chunk)], idx_v)
            pltpu.async_copy(table_hbm.at[idx_v], rows_v, sem).wait()
            pltpu.sync_copy(rows_v, out_hbm.at[pl.ds(off, chunk)])

    return k(table, idx)


def _attn_kernel(g_ref, qib_ref, sbb_ref, cqb_ref, y_ref):
    L, dk2 = g_ref.shape[2], g_ref.shape[3]
    dk = dk2 // 2
    r = _ROUNDS
    bl = _BL
    nb = L // bl

    def look_back2(x):
        prev = jnp.concatenate([x[nb - 1:], x[:nb - 1]], axis=0)
        return jnp.concatenate([prev, x], axis=1)

    def look_back3(x):
        prev = jnp.concatenate([x[nb - 1:], x[:nb - 1]], axis=0)
        return jnp.concatenate([prev, x], axis=1)

    G = g_ref[0, 0]
    rq = G[:, :dk]
    rv = G[:, dk:]
    kn = rq / jnp.maximum(
        jnp.sqrt(jnp.sum(rq * rq, axis=1, keepdims=True)), 1e-12)
    rq3 = rq.reshape(nb, bl, dk)
    rv3 = rv.reshape(nb, bl, dk)
    kn3 = kn.reshape(nb, bl, dk)
    lbk = look_back3(kn3)
    lbv = look_back3(rv3)
    qk = jax.lax.dot_general(
        rq3, lbk, (((2,), (2,)), ((0,), (0,))),
        preferred_element_type=jnp.float32) * (1.0 / math.sqrt(dk))
    sb3 = sbb_ref[0, 0]
    qi3 = qib_ref[0, 0]
    lbsb = look_back2(sb3)
    lbqi = look_back2(qi3)
    qk = jnp.where(sb3[:, :, None] != lbsb[:, None, :], -1e9, qk)
    qk = jnp.where(qi3[:, :, None] < lbqi[:, None, :], -1e9, qk)
    qk = jnp.where(qi3[:, :, None] == lbqi[:, None, :], -1e5, qk)
    ck = jnp.zeros((nb, bl, 2 * bl), jnp.float32)
    for r2 in range(r):
        cq = cqb_ref[0, 0, r2 * nb:(r2 + 1) * nb, :]
        ckey = look_back2(cq)
        cqm1 = jnp.where(cq == 0, nb - 1, cq - 1)
        hit = (ckey[:, None, :] == cq[:, :, None]) | (
            ckey[:, None, :] == cqm1[:, :, None])
        ck += hit.astype(jnp.float32)
    m = jnp.max(qk, axis=2)
    e = jnp.exp(qk - m[:, :, None])
    s = jnp.sum(e, axis=2)
    lse = m + jnp.log(s)
    sm = e / (s[:, :, None] * ck)
    att = jax.lax.dot_general(
        sm, lbv, (((2,), (1,)), ((0,), (0,))),
        preferred_element_type=jnp.float32)
    lm = jnp.max(lse, keepdims=True)
    w = jnp.exp(lse - lm)
    w = w / jnp.sum(w, keepdims=True)
    yw = (att * w[:, :, None]).reshape(L, dk)
    y_ref[0, 0] = jnp.concatenate([yw, jnp.zeros((L, dk), jnp.float32)], axis=1)


def _sum_kernel(y_ref, out_ref):
    dk = out_ref.shape[2]
    out_ref[0] = jnp.sum(y_ref[0, :, :, :dk], axis=0)


def kernel(query, value, seed, rand_matrix):
    B, L, dk = query.shape
    r = rand_matrix.shape[2]
    bl = _BL
    nb = L // bl
    qn = query / jnp.maximum(
        jnp.sqrt(jnp.sum(query * query, axis=-1, keepdims=True)), 1e-12)
    rm = rand_matrix / jnp.sqrt(
        jnp.sum(rand_matrix * rand_matrix, axis=1, keepdims=True))
    mm = jnp.einsum('bij,bjkl->bikl', qn, rm)
    h = jnp.argmax(jnp.concatenate([mm, -mm], -1), -1).astype(jnp.int32)
    h = h * L + jnp.arange(L, dtype=jnp.int32)[None, :, None]
    sorted_h = jnp.sort(h, axis=1)
    hi = (sorted_h % L).astype(jnp.int32)
    sb = (sorted_h // L).astype(jnp.int32)
    oi = jnp.argsort(hi, axis=1).astype(jnp.int32)
    ch = oi // bl
    chq = jnp.take_along_axis(
        jnp.broadcast_to(ch[:, :, None, :], (B, L, r, r)),
        jnp.broadcast_to(hi[:, :, :, None], (B, L, r, r)), axis=1)
    hi_t = jnp.transpose(hi, (0, 2, 1))
    oi_t = jnp.transpose(oi, (0, 2, 1))
    qib = hi_t.reshape(B, r, nb, bl)
    sbb = jnp.transpose(sb, (0, 2, 1)).reshape(B, r, nb, bl)
    cqb = jnp.transpose(chq, (0, 2, 3, 1)).reshape(B, r, r * nb, bl)
    qv = jnp.concatenate([query, value], axis=-1)

    boff = (jnp.arange(B, dtype=jnp.int32) * L)[:, None, None]
    gidx = (hi_t + boff).reshape(B * r * L)
    Gall = _sc_gather(qv.reshape(B * L, 2 * dk), gidx, 2 * dk)
    Gall = Gall.reshape(B, r, L, 2 * dk)

    y = pl.pallas_call(
        _attn_kernel,
        grid=(B, r),
        in_specs=[
            pl.BlockSpec((1, 1, L, 2 * dk), lambda b, rr: (b, rr, 0, 0)),
            pl.BlockSpec((1, 1, nb, bl), lambda b, rr: (b, rr, 0, 0)),
            pl.BlockSpec((1, 1, nb, bl), lambda b, rr: (b, rr, 0, 0)),
            pl.BlockSpec((1, 1, r * nb, bl), lambda b, rr: (b, rr, 0, 0)),
        ],
        out_specs=pl.BlockSpec((1, 1, L, 2 * dk), lambda b, rr: (b, rr, 0, 0)),
        out_shape=jax.ShapeDtypeStruct((B, r, L, 2 * dk), jnp.float32),
    )(Gall, qib, sbb, cqb)

    broff = (jnp.arange(B * r, dtype=jnp.int32) * L).reshape(B, r, 1)
    gidx2 = (oi_t + broff).reshape(B * r * L)
    yg = _sc_gather(y.reshape(B * r * L, 2 * dk), gidx2, 2 * dk)
    yg = yg.reshape(B, r, L, 2 * dk)

    out = pl.pallas_call(
        _sum_kernel,
        grid=(B,),
        in_specs=[pl.BlockSpec((1, r, L, 2 * dk), lambda b: (b, 0, 0, 0))],
        out_specs=pl.BlockSpec((1, L, dk), lambda b: (b, 0, 0)),
        out_shape=jax.ShapeDtypeStruct((B, L, dk), jnp.float32),
    )(yg)
    return out
